# baseline (speedup 1.0000x reference)
import functools

import jax
import jax.numpy as jnp
from jax import lax
from jax.experimental import pallas as pl
from jax.experimental.pallas import tpu as pltpu
from jax.experimental.pallas import tpu_sc as plsc

_BATCH = 16384
_INPUT_DIM = 39
_N_CONT = 13
_N_CAT = 26
_VOCAB = 100000
_EMB = 16
_OUT_DIM = _N_CONT + _N_CAT * _EMB
_TROWS = _N_CAT * _VOCAB // 8

_NC = 2
_NS = 16
_NW = _NC * _NS
_ROWS_PER_W = _BATCH // _NW
_CHUNK = 128
_N_CHUNKS = _ROWS_PER_W // _CHUNK
_NGRP = _CHUNK // 16


def _emb_kernel(x_hbm, tab_hbm, out_hbm, x_v, idx_v, rem_v, g_v, out_v, gsem, wsem):
    wid = lax.axis_index("s") * _NC + lax.axis_index("c")
    lane = lax.broadcasted_iota(jnp.int32, (16,), 0)
    off_a = lane * (_VOCAB // 8)
    off_b = (lane + 10) * (_VOCAB // 8)

    for t in range(_N_CHUNKS):
        base = wid * _ROWS_PER_W + t * _CHUNK

        pltpu.sync_copy(x_hbm.at[pl.ds(base, _CHUNK), :], x_v)

        def row_body(i, _):
            col_i = jnp.full((16,), i, jnp.int32)
            xa = x_v[i, pl.ds(_N_CONT, 16)]
            xb = x_v[i, pl.ds(_N_CONT + 10, 16)]
            plsc.store_scatter(idx_v, [lane, col_i], (xa >> 3) + off_a)
            plsc.store_scatter(idx_v, [lane + 10, col_i], (xb >> 3) + off_b)
            plsc.store_scatter(rem_v, [lane, col_i], (xa & 7) << 4)
            plsc.store_scatter(rem_v, [lane + 10, col_i], (xb & 7) << 4)
            xc = x_v[i, pl.ds(0, 16)]
            out_v[i, pl.ds(0, 16)] = xc.astype(jnp.float32)
            return 0

        lax.fori_loop(0, _CHUNK, row_body, 0)

        pltpu.make_async_copy(tab_hbm.at[idx_v.at[0]], g_v.at[0], gsem).start()

        def feat_body(c, _):
            buf = lax.rem(c, 2)
            pltpu.make_async_copy(
                tab_hbm.at[idx_v.at[c]], g_v.at[buf], gsem
            ).wait()

            @pl.when(c + 1 < _N_CAT)
            def _start_next():
                pltpu.make_async_copy(
                    tab_hbm.at[idx_v.at[c + 1]], g_v.at[lax.rem(c + 1, 2)], gsem
                ).start()

            bufv = jnp.full((16,), buf, jnp.int32)
            colbase = _N_CONT + _EMB * c

            def grp_body(g, _):
                rowv = g * 16 + lane
                r8 = rem_v[c, pl.ds(g * 16, 16)]
                for e in range(_EMB):
                    vals = plsc.load_gather(g_v, [bufv, rowv, r8 + e])
                    plsc.store_scatter(
                        out_v, [rowv, jnp.full((16,), colbase + e, jnp.int32)], vals
                    )
                return 0

            lax.fori_loop(0, _NGRP, grp_body, 0)
            return 0

        lax.fori_loop(0, _N_CAT, feat_body, 0)

        pltpu.sync_copy(out_v, out_hbm.at[pl.ds(base, _CHUNK), :])


@jax.jit
def _run(x, tab128):
    mesh = plsc.VectorSubcoreMesh(core_axis_name="c", subcore_axis_name="s")
    f = functools.partial(
        pl.kernel,
        mesh=mesh,
        out_type=jax.ShapeDtypeStruct((_BATCH, _OUT_DIM), jnp.float32),
        scratch_types=[
            pltpu.VMEM((_CHUNK, _INPUT_DIM), jnp.int32),
            pltpu.VMEM((_N_CAT, _CHUNK), jnp.int32),
            pltpu.VMEM((_N_CAT, _CHUNK), jnp.int32),
            pltpu.VMEM((2, _CHUNK, 128), jnp.float32),
            pltpu.VMEM((_CHUNK, _OUT_DIM), jnp.float32),
            pltpu.SemaphoreType.DMA,
            pltpu.SemaphoreType.DMA,
        ],
        compiler_params=pltpu.CompilerParams(
            use_tc_tiling_on_sc=True, needs_layout_passes=False
        ),
    )(_emb_kernel)
    return f(x, tab128)


def kernel(x, tables):
    tab128 = tables.reshape(_TROWS, 128)
    return _run(x, tab128)
